# baseline (speedup 1.0000x reference)
import functools

import jax
import jax.numpy as jnp
from jax.experimental import pallas as pl
from jax.experimental.pallas import tpu as pltpu

H = 12
N_CAND = 1024
TOP_K = 128
ITERS = 6
A_DIM = 64
D_STATE = 256
CACHE = 1000
PROP_MIN_STD = 0.05
MIN_STD = 0.02
NBR = 64
REP = N_CAND // NBR

_DOT = functools.partial(jnp.dot, preferred_element_type=jnp.float32,
                         precision=jax.lax.Precision.DEFAULT)


def _gather_body(nbr_ref, cm_ref, cs_ref, gm_ref, gs_ref):
    def step(k, _):
        idx = nbr_ref[k]
        gm_ref[pl.ds(k, 1), :] = cm_ref[pl.ds(idx, 1), :]
        gs_ref[pl.ds(k, 1), :] = jnp.maximum(cs_ref[pl.ds(idx, 1), :], PROP_MIN_STD)
        return 0

    jax.lax.fori_loop(0, NBR, step, 0)


def _select_mask(cost):
    M = cost.shape[0]
    R = M // 128
    bits = jax.lax.bitcast_convert_type(cost, jnp.int32).reshape(R, 128)
    idx = (jax.lax.broadcasted_iota(jnp.int32, (R, 128), 0) * 128
           + jax.lax.broadcasted_iota(jnp.int32, (R, 128), 1))

    def tbody(_, lh):
        lo, hi = lh
        mid = lo + (hi - lo) // 2
        cnt = jnp.sum(jnp.where(bits <= mid, 1, 0))
        pred = cnt >= TOP_K
        return jnp.where(pred, lo, mid + 1), jnp.where(pred, mid, hi)

    t_lo, _ = jax.lax.fori_loop(
        0, 31, tbody, (jnp.int32(0), jnp.int32(2**31 - 1)))
    c_lt = jnp.sum(jnp.where(bits < t_lo, 1, 0))
    need = TOP_K - c_lt
    midx = jnp.where(bits == t_lo, idx, jnp.int32(M))

    def ibody(_, lh):
        lo, hi = lh
        mid = lo + (hi - lo) // 2
        cnt = jnp.sum(jnp.where(midx <= mid, 1, 0))
        pred = cnt >= need
        return jnp.where(pred, lo, mid + 1), jnp.where(pred, mid, hi)

    j_lo, _ = jax.lax.fori_loop(
        0, 12, ibody, (jnp.int32(0), jnp.int32(M - 1)))
    mask = ((bits < t_lo) | ((bits == t_lo) & (idx <= j_lo)))
    return mask.astype(jnp.float32).reshape(M, 1)


def _cem_body(ra_ref, rp_ref, gm_ref, gs_ref, wd_ref, wa_ref,
              q_ref, init_ref, c_ref, r_ref, out_ref, mean_s, std_s):
    i = pl.program_id(0)
    wd = wd_ref[:]
    wa = wa_ref[:]
    qv = q_ref[:]
    init = init_ref[:]
    center = c_ref[:]
    half = r_ref[:] * 0.5
    ra = ra_ref[0]

    def rollout_and_refit(make_clamped, M):
        z0 = _DOT(init, wd)
        cost = None
        s = None
        for t in range(H):
            act = make_clamped(t) * half + center
            za = _DOT(act, wa)
            s = jnp.tanh((z0 if t == 0 else _DOT(s, wd)) + za)
            c_t = jnp.sum((s * s) * qv, axis=1, keepdims=True)
            cost = c_t if cost is None else cost + c_t
        mask = _select_mask(cost)
        inv_k = 1.0 / float(TOP_K)
        for t in range(H):
            c_t = make_clamped(t)
            mean_t = jnp.sum(c_t * mask, axis=0, keepdims=True) * inv_k
            dev = (c_t - mean_t)
            var_t = jnp.sum(dev * dev * mask, axis=0, keepdims=True) * inv_k
            std_t = jnp.maximum(jnp.sqrt(var_t), MIN_STD)
            mean_s[t:t + 1, :] = mean_t
            std_s[t:t + 1, :] = std_t
            out_ref[t:t + 1, :] = mean_t * half + center

    @pl.when(i == 0)
    def _first_iter():
        def make_clamped(t):
            gm_t = jnp.broadcast_to(gm_ref[:, t, :][None], (REP, NBR, A_DIM)).reshape(N_CAND, A_DIM)
            gs_t = jnp.broadcast_to(gs_ref[:, t, :][None], (REP, NBR, A_DIM)).reshape(N_CAND, A_DIM)
            prop = gm_t + rp_ref[t] * gs_t
            prop_n = (prop - center) / half
            a_n = jnp.concatenate([ra[t], prop_n], axis=0)
            return jnp.clip(a_n, -1.0, 1.0)

        rollout_and_refit(make_clamped, 2 * N_CAND)

    @pl.when(i > 0)
    def _later_iters():
        def make_clamped(t):
            a_n = mean_s[t:t + 1, :] + std_s[t:t + 1, :] * ra[t]
            return jnp.clip(a_n, -1.0, 1.0)

        rollout_and_refit(make_clamped, N_CAND)


def kernel(neighbor_states, cache_means, cache_stds, act_center, act_range,
           W_dyn, W_act, q, init_state):
    key = jax.random.key(42)
    ra_list = []
    rand_prop = None
    for i in range(ITERS):
        key, k1, k2 = jax.random.split(key, 3)
        ra_list.append(jax.random.normal(k1, (H, N_CAND, A_DIM), dtype=jnp.float32))
        if i == 0:
            rand_prop = jax.random.normal(k2, (H, N_CAND, A_DIM), dtype=jnp.float32)
    rand_act = jnp.stack(ra_list)

    nbr1d = neighbor_states.astype(jnp.int32)
    cmF = cache_means.reshape(CACHE, H * A_DIM)
    csF = cache_stds.reshape(CACHE, H * A_DIM)
    q2 = q.reshape(1, D_STATE)
    init2 = init_state.reshape(1, D_STATE)
    c2 = act_center.reshape(1, A_DIM)
    r2 = act_range.reshape(1, A_DIM)

    gm, gs = pl.pallas_call(
        _gather_body,
        in_specs=[
            pl.BlockSpec(memory_space=pltpu.SMEM),
            pl.BlockSpec(memory_space=pltpu.VMEM),
            pl.BlockSpec(memory_space=pltpu.VMEM),
        ],
        out_shape=[
            jax.ShapeDtypeStruct((NBR, H * A_DIM), jnp.float32),
            jax.ShapeDtypeStruct((NBR, H * A_DIM), jnp.float32),
        ],
    )(nbr1d, cmF, csF)
    gm = gm.reshape(NBR, H, A_DIM)
    gs = gs.reshape(NBR, H, A_DIM)

    out = pl.pallas_call(
        _cem_body,
        grid=(ITERS,),
        in_specs=[
            pl.BlockSpec((1, H, N_CAND, A_DIM), lambda i: (i, 0, 0, 0)),
            pl.BlockSpec((H, N_CAND, A_DIM), lambda i: (0, 0, 0)),
            pl.BlockSpec((NBR, H, A_DIM), lambda i: (0, 0, 0)),
            pl.BlockSpec((NBR, H, A_DIM), lambda i: (0, 0, 0)),
            pl.BlockSpec((D_STATE, D_STATE), lambda i: (0, 0)),
            pl.BlockSpec((A_DIM, D_STATE), lambda i: (0, 0)),
            pl.BlockSpec((1, D_STATE), lambda i: (0, 0)),
            pl.BlockSpec((1, D_STATE), lambda i: (0, 0)),
            pl.BlockSpec((1, A_DIM), lambda i: (0, 0)),
            pl.BlockSpec((1, A_DIM), lambda i: (0, 0)),
        ],
        out_specs=pl.BlockSpec((H, A_DIM), lambda i: (0, 0)),
        out_shape=jax.ShapeDtypeStruct((H, A_DIM), jnp.float32),
        scratch_shapes=[
            pltpu.VMEM((H, A_DIM), jnp.float32),
            pltpu.VMEM((H, A_DIM), jnp.float32),
        ],
    )(rand_act, rand_prop, gm, gs, W_dyn, W_act, q2, init2, c2, r2)
    return out


# speedup vs baseline: 1.1482x; 1.1482x over previous
import functools

import jax
import jax.numpy as jnp
from jax.experimental import pallas as pl
from jax.experimental.pallas import tpu as pltpu

H = 12
N_CAND = 1024
TOP_K = 128
ITERS = 6
A_DIM = 64
D_STATE = 256
CACHE = 1000
PROP_MIN_STD = 0.05
MIN_STD = 0.02
NBR = 64
REP = N_CAND // NBR

_DOT = functools.partial(jnp.dot, preferred_element_type=jnp.float32,
                         precision=jax.lax.Precision.DEFAULT)


def _gather_body(nbr_ref, cm_ref, cs_ref, gm_ref, gs_ref):
    def step(k, _):
        idx = nbr_ref[k]
        gm_ref[pl.ds(k, 1), :] = cm_ref[pl.ds(idx, 1), :]
        gs_ref[pl.ds(k, 1), :] = jnp.maximum(cs_ref[pl.ds(idx, 1), :], PROP_MIN_STD)
        return 0

    jax.lax.fori_loop(0, NBR, step, 0)


def _select_mask(cost):
    M = cost.shape[0]
    cost_row = jnp.transpose(cost)
    row_ids = jax.lax.broadcasted_iota(jnp.int32, (M, 1), 0)
    ones = jnp.ones((M, 1), dtype=jnp.float32)
    chunks = []
    CH = 256
    for base in range(0, M, CH):
        c_i = jax.lax.slice(cost, (base, 0), (base + CH, 1))
        i_i = jax.lax.slice(row_ids, (base, 0), (base + CH, 1))
        j_ids = jax.lax.broadcasted_iota(jnp.int32, (CH, M), 1)
        before = (cost_row < c_i) | ((cost_row == c_i) & (j_ids < i_i))
        rank = _DOT(before.astype(jnp.float32), ones)
        chunks.append((rank < float(TOP_K)).astype(jnp.float32))
    return jnp.concatenate(chunks, axis=0)


def _cem_body(ra_ref, rp_ref, gm_ref, gs_ref, wd_ref, wa_ref,
              q_ref, init_ref, c_ref, r_ref, out_ref, mean_s, std_s):
    i = pl.program_id(0)
    wd = wd_ref[:]
    wa = wa_ref[:]
    qv = q_ref[:]
    init = init_ref[:]
    center = c_ref[:]
    half = r_ref[:] * 0.5
    ra = ra_ref[0]

    def rollout_and_refit(make_clamped, M):
        z0 = _DOT(init, wd)
        cost = None
        s = None
        for t in range(H):
            act = make_clamped(t) * half + center
            za = _DOT(act, wa)
            s = jnp.tanh((z0 if t == 0 else _DOT(s, wd)) + za)
            c_t = jnp.sum((s * s) * qv, axis=1, keepdims=True)
            cost = c_t if cost is None else cost + c_t
        mask = _select_mask(cost)
        inv_k = 1.0 / float(TOP_K)
        for t in range(H):
            c_t = make_clamped(t)
            mean_t = jnp.sum(c_t * mask, axis=0, keepdims=True) * inv_k
            dev = (c_t - mean_t)
            var_t = jnp.sum(dev * dev * mask, axis=0, keepdims=True) * inv_k
            std_t = jnp.maximum(jnp.sqrt(var_t), MIN_STD)
            mean_s[t:t + 1, :] = mean_t
            std_s[t:t + 1, :] = std_t
            out_ref[t:t + 1, :] = mean_t * half + center

    @pl.when(i == 0)
    def _first_iter():
        def make_clamped(t):
            gm_t = jnp.broadcast_to(gm_ref[:, t, :][None], (REP, NBR, A_DIM)).reshape(N_CAND, A_DIM)
            gs_t = jnp.broadcast_to(gs_ref[:, t, :][None], (REP, NBR, A_DIM)).reshape(N_CAND, A_DIM)
            prop = gm_t + rp_ref[t] * gs_t
            prop_n = (prop - center) / half
            a_n = jnp.concatenate([ra[t], prop_n], axis=0)
            return jnp.clip(a_n, -1.0, 1.0)

        rollout_and_refit(make_clamped, 2 * N_CAND)

    @pl.when(i > 0)
    def _later_iters():
        def make_clamped(t):
            a_n = mean_s[t:t + 1, :] + std_s[t:t + 1, :] * ra[t]
            return jnp.clip(a_n, -1.0, 1.0)

        rollout_and_refit(make_clamped, N_CAND)


def kernel(neighbor_states, cache_means, cache_stds, act_center, act_range,
           W_dyn, W_act, q, init_state):
    key = jax.random.key(42)
    ra_list = []
    rand_prop = None
    for i in range(ITERS):
        key, k1, k2 = jax.random.split(key, 3)
        ra_list.append(jax.random.normal(k1, (H, N_CAND, A_DIM), dtype=jnp.float32))
        if i == 0:
            rand_prop = jax.random.normal(k2, (H, N_CAND, A_DIM), dtype=jnp.float32)
    rand_act = jnp.stack(ra_list)

    nbr1d = neighbor_states.astype(jnp.int32)
    cmF = cache_means.reshape(CACHE, H * A_DIM)
    csF = cache_stds.reshape(CACHE, H * A_DIM)
    q2 = q.reshape(1, D_STATE)
    init2 = init_state.reshape(1, D_STATE)
    c2 = act_center.reshape(1, A_DIM)
    r2 = act_range.reshape(1, A_DIM)

    gm, gs = pl.pallas_call(
        _gather_body,
        in_specs=[
            pl.BlockSpec(memory_space=pltpu.SMEM),
            pl.BlockSpec(memory_space=pltpu.VMEM),
            pl.BlockSpec(memory_space=pltpu.VMEM),
        ],
        out_shape=[
            jax.ShapeDtypeStruct((NBR, H * A_DIM), jnp.float32),
            jax.ShapeDtypeStruct((NBR, H * A_DIM), jnp.float32),
        ],
    )(nbr1d, cmF, csF)
    gm = gm.reshape(NBR, H, A_DIM)
    gs = gs.reshape(NBR, H, A_DIM)

    out = pl.pallas_call(
        _cem_body,
        grid=(ITERS,),
        in_specs=[
            pl.BlockSpec((1, H, N_CAND, A_DIM), lambda i: (i, 0, 0, 0)),
            pl.BlockSpec((H, N_CAND, A_DIM), lambda i: (0, 0, 0)),
            pl.BlockSpec((NBR, H, A_DIM), lambda i: (0, 0, 0)),
            pl.BlockSpec((NBR, H, A_DIM), lambda i: (0, 0, 0)),
            pl.BlockSpec((D_STATE, D_STATE), lambda i: (0, 0)),
            pl.BlockSpec((A_DIM, D_STATE), lambda i: (0, 0)),
            pl.BlockSpec((1, D_STATE), lambda i: (0, 0)),
            pl.BlockSpec((1, D_STATE), lambda i: (0, 0)),
            pl.BlockSpec((1, A_DIM), lambda i: (0, 0)),
            pl.BlockSpec((1, A_DIM), lambda i: (0, 0)),
        ],
        out_specs=pl.BlockSpec((H, A_DIM), lambda i: (0, 0)),
        out_shape=jax.ShapeDtypeStruct((H, A_DIM), jnp.float32),
        scratch_shapes=[
            pltpu.VMEM((H, A_DIM), jnp.float32),
            pltpu.VMEM((H, A_DIM), jnp.float32),
        ],
    )(rand_act, rand_prop, gm, gs, W_dyn, W_act, q2, init2, c2, r2)
    return out


# speedup vs baseline: 1.2473x; 1.0864x over previous
import functools

import jax
import jax.numpy as jnp
from jax.experimental import pallas as pl
from jax.experimental.pallas import tpu as pltpu

H = 12
N_CAND = 1024
TOP_K = 128
ITERS = 6
A_DIM = 64
D_STATE = 256
CACHE = 1000
PROP_MIN_STD = 0.05
MIN_STD = 0.02
NBR = 64
REP = N_CAND // NBR

_DOT = functools.partial(jnp.dot, preferred_element_type=jnp.float32,
                         precision=jax.lax.Precision.DEFAULT)


def _gather_body(nbr_ref, cm_ref, cs_ref, gm_ref, gs_ref):
    def step(k, _):
        idx = nbr_ref[k]
        gm_ref[pl.ds(k, 1), :] = cm_ref[pl.ds(idx, 1), :]
        gs_ref[pl.ds(k, 1), :] = jnp.maximum(cs_ref[pl.ds(idx, 1), :], PROP_MIN_STD)
        return 0

    jax.lax.fori_loop(0, NBR, step, 0)


def _select_mask(cost):
    M = cost.shape[0]
    cost_row = jnp.transpose(cost)
    row_ids = jax.lax.broadcasted_iota(jnp.int32, (M, 1), 0)
    ones = jnp.ones((M, 1), dtype=jnp.float32)
    chunks = []
    CH = 256
    for base in range(0, M, CH):
        c_i = jax.lax.slice(cost, (base, 0), (base + CH, 1))
        i_i = jax.lax.slice(row_ids, (base, 0), (base + CH, 1))
        j_ids = jax.lax.broadcasted_iota(jnp.int32, (CH, M), 1)
        before = (cost_row < c_i) | ((cost_row == c_i) & (j_ids < i_i))
        rank = _DOT(before.astype(jnp.float32), ones)
        chunks.append((rank < float(TOP_K)).astype(jnp.float32))
    return jnp.concatenate(chunks, axis=0)


def _cem_body(ra_ref, rp_ref, gm_ref, gs_ref, wd_ref, wa_ref,
              q_ref, init_ref, c_ref, r_ref, out_ref, mean_s, std_s, clamp_s):
    i = pl.program_id(0)
    wd = wd_ref[:]
    wa = wa_ref[:]
    qv = q_ref[:]
    init = init_ref[:]
    center = c_ref[:]
    half = r_ref[:] * 0.5
    ra = ra_ref[0]

    def rollout_and_refit(make_clamped, M):
        z0 = _DOT(init, wd)
        cost = None
        s = None
        for t in range(H):
            c_t = make_clamped(t)
            clamp_s[t, 0:M, :] = c_t
            act = c_t * half + center
            za = _DOT(act, wa)
            s = jnp.tanh((z0 if t == 0 else _DOT(s, wd)) + za)
            c_t = jnp.sum((s * s) * qv, axis=1, keepdims=True)
            cost = c_t if cost is None else cost + c_t
        mask = _select_mask(cost)
        inv_k = 1.0 / float(TOP_K)
        for t in range(H):
            c_t = clamp_s[t, 0:M, :]
            mean_t = jnp.sum(c_t * mask, axis=0, keepdims=True) * inv_k
            dev = (c_t - mean_t)
            var_t = jnp.sum(dev * dev * mask, axis=0, keepdims=True) * inv_k
            std_t = jnp.maximum(jnp.sqrt(var_t), MIN_STD)
            mean_s[t:t + 1, :] = mean_t
            std_s[t:t + 1, :] = std_t
            out_ref[t:t + 1, :] = mean_t * half + center

    @pl.when(i == 0)
    def _first_iter():
        def make_clamped(t):
            gm_t = jnp.broadcast_to(gm_ref[:, t, :][None], (REP, NBR, A_DIM)).reshape(N_CAND, A_DIM)
            gs_t = jnp.broadcast_to(gs_ref[:, t, :][None], (REP, NBR, A_DIM)).reshape(N_CAND, A_DIM)
            prop = gm_t + rp_ref[t] * gs_t
            prop_n = (prop - center) / half
            a_n = jnp.concatenate([ra[t], prop_n], axis=0)
            return jnp.clip(a_n, -1.0, 1.0)

        rollout_and_refit(make_clamped, 2 * N_CAND)

    @pl.when(i > 0)
    def _later_iters():
        def make_clamped(t):
            a_n = mean_s[t:t + 1, :] + std_s[t:t + 1, :] * ra[t]
            return jnp.clip(a_n, -1.0, 1.0)

        rollout_and_refit(make_clamped, N_CAND)


def kernel(neighbor_states, cache_means, cache_stds, act_center, act_range,
           W_dyn, W_act, q, init_state):
    key = jax.random.key(42)
    ra_list = []
    rand_prop = None
    for i in range(ITERS):
        key, k1, k2 = jax.random.split(key, 3)
        ra_list.append(jax.random.normal(k1, (H, N_CAND, A_DIM), dtype=jnp.float32))
        if i == 0:
            rand_prop = jax.random.normal(k2, (H, N_CAND, A_DIM), dtype=jnp.float32)
    rand_act = jnp.stack(ra_list)

    nbr1d = neighbor_states.astype(jnp.int32)
    cmF = cache_means.reshape(CACHE, H * A_DIM)
    csF = cache_stds.reshape(CACHE, H * A_DIM)
    q2 = q.reshape(1, D_STATE)
    init2 = init_state.reshape(1, D_STATE)
    c2 = act_center.reshape(1, A_DIM)
    r2 = act_range.reshape(1, A_DIM)

    gm, gs = pl.pallas_call(
        _gather_body,
        in_specs=[
            pl.BlockSpec(memory_space=pltpu.SMEM),
            pl.BlockSpec(memory_space=pltpu.VMEM),
            pl.BlockSpec(memory_space=pltpu.VMEM),
        ],
        out_shape=[
            jax.ShapeDtypeStruct((NBR, H * A_DIM), jnp.float32),
            jax.ShapeDtypeStruct((NBR, H * A_DIM), jnp.float32),
        ],
    )(nbr1d, cmF, csF)
    gm = gm.reshape(NBR, H, A_DIM)
    gs = gs.reshape(NBR, H, A_DIM)

    out = pl.pallas_call(
        _cem_body,
        grid=(ITERS,),
        in_specs=[
            pl.BlockSpec((1, H, N_CAND, A_DIM), lambda i: (i, 0, 0, 0)),
            pl.BlockSpec((H, N_CAND, A_DIM), lambda i: (0, 0, 0)),
            pl.BlockSpec((NBR, H, A_DIM), lambda i: (0, 0, 0)),
            pl.BlockSpec((NBR, H, A_DIM), lambda i: (0, 0, 0)),
            pl.BlockSpec((D_STATE, D_STATE), lambda i: (0, 0)),
            pl.BlockSpec((A_DIM, D_STATE), lambda i: (0, 0)),
            pl.BlockSpec((1, D_STATE), lambda i: (0, 0)),
            pl.BlockSpec((1, D_STATE), lambda i: (0, 0)),
            pl.BlockSpec((1, A_DIM), lambda i: (0, 0)),
            pl.BlockSpec((1, A_DIM), lambda i: (0, 0)),
        ],
        out_specs=pl.BlockSpec((H, A_DIM), lambda i: (0, 0)),
        out_shape=jax.ShapeDtypeStruct((H, A_DIM), jnp.float32),
        scratch_shapes=[
            pltpu.VMEM((H, A_DIM), jnp.float32),
            pltpu.VMEM((H, A_DIM), jnp.float32),
            pltpu.VMEM((H, 2 * N_CAND, A_DIM), jnp.float32),
        ],
    )(rand_act, rand_prop, gm, gs, W_dyn, W_act, q2, init2, c2, r2)
    return out
